# baseline (speedup 1.0000x reference)
import functools

import jax
import jax.numpy as jnp
from jax import lax
from jax.experimental import pallas as pl
from jax.experimental.pallas import tpu as pltpu
from jax.experimental.pallas import tpu_sc as plsc

N = 10000
E = 320000
D = 128
B = 64
NC = 2
NS = 16
NW = NC * NS
EPW = E // NW
C = 80
NCHUNK = EPW // C
L = 16

def _dot(a, b, precision=None):
    return jnp.dot(a, b, preferred_element_type=jnp.float32,
                   precision=precision)


def _make_sc_edge():
    mesh = plsc.VectorSubcoreMesh(core_axis_name="c", subcore_axis_name="s",
                                  num_cores=NC, num_subcores=NS)

    @functools.partial(
        pl.kernel,
        out_type=(jax.ShapeDtypeStruct((NC, N, D), jnp.float32),
                  jax.ShapeDtypeStruct((NW, N), jnp.float32)),
        mesh=mesh,
        compiler_params=pltpu.CompilerParams(needs_layout_passes=False),
        scratch_types=[
            pltpu.VMEM((N,), jnp.float32),
            pltpu.VMEM((N,), jnp.float32),
            pltpu.VMEM((N,), jnp.float32),
            pltpu.VMEM((C,), jnp.int32),
            pltpu.VMEM((C,), jnp.int32),
            pltpu.VMEM((C + L,), jnp.float32),
            pltpu.VMEM((L,), jnp.int32),
            pltpu.VMEM((L,), jnp.float32),
            pltpu.VMEM((C, D), jnp.float32),
            pltpu.VMEM_SHARED((N, D), jnp.float32),
            pltpu.SemaphoreType.DMA,
        ],
    )
    def sc_edge(hx_hbm, s2_hbm, src_hbm, dst_hbm, zn_hbm, num_out, den_out,
                ea2_t, eb2_t, den_t, sidx, didx, exb, skbuf,
                csbuf, rows, num_sh, sem):
        c = lax.axis_index("c")
        s = lax.axis_index("s")
        wid = s * NC + c

        pltpu.sync_copy(s2_hbm.at[0], ea2_t)
        pltpu.sync_copy(s2_hbm.at[1], eb2_t)

        z16 = jnp.zeros((L,), jnp.float32)

        def zden(i, carry):
            den_t[pl.ds(i * L, L)] = z16
            return carry

        lax.fori_loop(0, N // L, zden, 0)

        @pl.when(s == 0)
        def _():
            pltpu.sync_copy(zn_hbm, num_sh)

        plsc.subcore_barrier()

        lanes = lax.iota(jnp.int32, L)
        nxt = jnp.minimum(lanes + 1, L - 1)
        prv = jnp.maximum(lanes - 1, 0)

        def chunk(i, carry):
            base = wid * EPW + i * C
            pltpu.sync_copy(src_hbm.at[pl.ds(base, C)], sidx)
            pltpu.sync_copy(dst_hbm.at[pl.ds(base, C)], didx)
            pltpu.async_copy(hx_hbm.at[sidx], rows, sem).wait()
            for g in range(C // L):
                iv = sidx[pl.ds(g * L, L)]
                dv = didx[pl.ds(g * L, L)]
                p2 = (plsc.load_gather(ea2_t, [iv])
                      * plsc.load_gather(eb2_t, [dv]))
                q = p2 * p2
                ex = jnp.maximum(q * q * p2, p2)
                exb[pl.ds(g * L, L)] = ex
                if g == 0:
                    exb[pl.ds(C, L)] = ex
                sk, sx = plsc.sort_key_val(dv, ex)
                cs = plsc.cumsum(sx)
                skbuf[...] = sk
                csbuf[...] = cs
                sk_next = plsc.load_gather(skbuf, [nxt])
                sk_prev = plsc.load_gather(skbuf, [prv])
                m_end = (sk != sk_next) | (lanes == L - 1)
                r_start = (sk != sk_prev) | (lanes == 0)
                idx_start = plsc.cummax(jnp.where(r_start, lanes, 0))
                cs_prev = plsc.load_gather(
                    csbuf, [jnp.maximum(idx_start - 1, 0)])
                runsum = cs - jnp.where(idx_start > 0, cs_prev, 0.0)
                plsc.addupdate_scatter(den_t, [sk], runsum, mask=m_end)
            for r in range(C):
                ri = C if r == 0 else r
                scv = plsc.load_gather(
                    exb, [jnp.full((L,), ri, jnp.int32)])
                for j in range(D // L):
                    rows[r, pl.ds(j * L, L)] = rows[r, pl.ds(j * L, L)] * scv
            pltpu.sync_copy(rows, num_sh.at[didx], add=True)
            return carry

        lax.fori_loop(0, NCHUNK, chunk, 0)
        pltpu.sync_copy(den_t, den_out.at[wid])
        plsc.subcore_barrier()

        @pl.when(s == 0)
        def _():
            pltpu.sync_copy(num_sh, num_out.at[c])

    return sc_edge


_sc_edge = _make_sc_edge()


def _elu(v):
    return jnp.where(v > 0.0, v, jnp.exp(jnp.minimum(v, 0.0)) - 1.0)


def _norm_act(num, denp):
    tot = num[0] + num[1]
    den = lax.dot_general(denp, jnp.ones((NW, 1), jnp.float32),
                          (((0,), (0,)), ((), ())),
                          preferred_element_type=jnp.float32,
                          precision=lax.Precision.HIGHEST)
    act = tot / (den + 1e-16)
    return _elu(act)


def _logits(a2, h):
    s2 = lax.dot_general(a2, h, (((1,), (1,)), ((), ())),
                         preferred_element_type=jnp.float32,
                         precision=lax.Precision.HIGHEST)
    return jnp.exp(0.2 * s2)


def _tc_first_body(x_ref, w_ref, a2_ref, hx_ref, s2_ref):
    h = _dot(x_ref[...], w_ref[...])
    hx_ref[...] = h
    s2_ref[...] = _logits(a2_ref[...], h)


def _tc_mid_body(num_ref, denp_ref, w_ref, a2_ref, hx_ref, s2_ref):
    act = _norm_act(num_ref[...], denp_ref[...])
    h = _dot(act, w_ref[...])
    hx_ref[...] = h
    s2_ref[...] = _logits(a2_ref[...], h)


def _tc_cat_body(num_ref, denp_ref, x_ref, wa_ref, wb_ref, a2_ref,
                 hx_ref, s2_ref):
    act = _norm_act(num_ref[...], denp_ref[...])
    h = (_dot(x_ref[...], wa_ref[...])
         + _dot(act, wb_ref[...]))
    hx_ref[...] = h
    s2_ref[...] = _logits(a2_ref[...], h)


_layer_out = (jax.ShapeDtypeStruct((N, D), jnp.float32),
              jax.ShapeDtypeStruct((2, N), jnp.float32))

_tc_first = pl.pallas_call(_tc_first_body, out_shape=_layer_out)
_tc_mid = pl.pallas_call(_tc_mid_body, out_shape=_layer_out)
_tc_cat = pl.pallas_call(_tc_cat_body, out_shape=_layer_out)


def _tc_final_body(num_ref, denp_ref, batch_ref, mf_ref,
                   mw1_ref, mb1_ref, mw2_ref, mb2_ref,
                   wq_ref, wk_ref, wv_ref,
                   pw1_ref, pb1_ref, pw2_ref, pb2_ref, pw3_ref, pb3_ref,
                   pw4_ref, pb4_ref, pw5_ref, pb5_ref,
                   rw1_ref, rb1_ref, rw2_ref, rb2_ref, rw3_ref, rb3_ref,
                   rw4_ref, rb4_ref, rw5_ref, rb5_ref,
                   energy_ref, pi_ref):
    h = _norm_act(num_ref[...], denp_ref[...])
    batch = batch_ref[...]
    onehot = (batch == lax.broadcasted_iota(jnp.int32, (1, B), 1)
              ).astype(jnp.float32)

    def rdot(m, v):
        return lax.dot_general(m, v, (((0,), (0,)), ((), ())),
                               preferred_element_type=jnp.float32,
                               precision=None)

    cnt = rdot(onehot, jnp.ones((N, 1), jnp.float32))
    pooled = rdot(onehot, h) / jnp.maximum(cnt, 1.0)

    z = jax.nn.relu(_dot(pooled, pw1_ref[...]) + pb1_ref[...])
    z = jax.nn.relu(_dot(z, pw2_ref[...]) + pb2_ref[...])
    z = jax.nn.relu(_dot(z, pw3_ref[...]) + pb3_ref[...])
    z = jax.nn.relu(_dot(z, pw4_ref[...]) + pb4_ref[...])
    pi_out = _dot(z, pw5_ref[...]) + pb5_ref[...]

    m1 = jax.nn.relu(_dot(mf_ref[...], mw1_ref[...]) + mb1_ref[...])
    metal = jax.nn.relu(_dot(m1, mw2_ref[...]) + mb2_ref[...])

    q = _dot(metal, wq_ref[...])
    k = _dot(h, wk_ref[...], None)
    v = _dot(h, wv_ref[...], None)
    qn = _dot(onehot, q, None)
    scores = jnp.sum(qn * k, axis=1, keepdims=True) / jnp.sqrt(
        jnp.float32(D))
    mseg = jnp.max(jnp.where(onehot > 0.0, scores, -1e30), axis=0,
                   keepdims=True)
    mn = _dot(onehot, mseg.T, None)
    exn = jnp.exp(scores - mn)
    denb = rdot(onehot, exn)
    dn = _dot(onehot, denb, None)
    w = exn / (dn + 1e-16)
    attn = rdot(onehot * w, v)

    cfeat = jnp.concatenate([pi_out, attn, metal], axis=1)
    r = jax.nn.relu(_dot(cfeat, rw1_ref[...]) + rb1_ref[...])
    r = jax.nn.relu(_dot(r, rw2_ref[...]) + rb2_ref[...])
    r = jax.nn.relu(_dot(r, rw3_ref[...]) + rb3_ref[...])
    r = jax.nn.relu(_dot(r, rw4_ref[...]) + rb4_ref[...])
    energy_ref[...] = _dot(r, rw5_ref[...]) + rb5_ref[...]
    pi_ref[...] = pi_out


_tc_final = pl.pallas_call(
    _tc_final_body,
    out_shape=(jax.ShapeDtypeStruct((B, 1), jnp.float32),
               jax.ShapeDtypeStruct((B, 10), jnp.float32)),
)


def kernel(x, edge_index, batch, metal_features,
           Wn0, Wn, an, Wg0, Wg, ag,
           mw1, mb1, mw2, mb2,
           wq, wk, wv,
           pw1, pb1, pw2, pb2, pw3, pb3, pw4, pb4, pw5, pb5,
           rw1, rb1, rw2, rb2, rw3, rb3, rw4, rb4, rw5, rb5):
    zeros = jnp.zeros((N, D), jnp.float32)

    src_e = edge_index[0]
    dst_e = edge_index[1]
    hx, s2 = _tc_first(x, Wn0, an[0])
    num, denp = _sc_edge(hx, s2, src_e, dst_e, zeros)
    for i in range(3):
        hx, s2 = _tc_mid(num, denp, Wn[i], an[i + 1])
        num, denp = _sc_edge(hx, s2, src_e, dst_e, zeros)
    hx, s2 = _tc_cat(num, denp, x, Wg0[:D], Wg0[D:], ag[0])
    num, denp = _sc_edge(hx, s2, src_e, dst_e, zeros)
    for i in range(3):
        hx, s2 = _tc_mid(num, denp, Wg[i], ag[i + 1])
        num, denp = _sc_edge(hx, s2, src_e, dst_e, zeros)

    energy, pi_out = _tc_final(
        num, denp, batch.reshape(N, 1), metal_features,
        mw1, mb1, mw2, mb2, wq, wk, wv,
        pw1, pb1, pw2, pb2, pw3, pb3, pw4, pb4, pw5, pb5,
        rw1, rb1, rw2, rb2, rw3, rb3, rw4, rb4, rw5, rb5)
    return energy[:, 0], pi_out


# speedup vs baseline: 1.0666x; 1.0666x over previous
import functools

import jax
import jax.numpy as jnp
from jax import lax
from jax.experimental import pallas as pl
from jax.experimental.pallas import tpu as pltpu
from jax.experimental.pallas import tpu_sc as plsc

N = 10000
E = 320000
D = 128
B = 64
NC = 2
NS = 16
NW = NC * NS
EPW = E // NW
C = 80
NCHUNK = EPW // C
L = 16

def _dot(a, b, precision=None):
    return jnp.dot(a, b, preferred_element_type=jnp.float32,
                   precision=precision)


def _make_sc_edge():
    mesh = plsc.VectorSubcoreMesh(core_axis_name="c", subcore_axis_name="s",
                                  num_cores=NC, num_subcores=NS)

    @functools.partial(
        pl.kernel,
        out_type=(jax.ShapeDtypeStruct((NC, N, D), jnp.float32),
                  jax.ShapeDtypeStruct((NW, N), jnp.float32)),
        mesh=mesh,
        compiler_params=pltpu.CompilerParams(needs_layout_passes=False),
        scratch_types=[
            pltpu.VMEM((N,), jnp.float32),
            pltpu.VMEM((N,), jnp.float32),
            pltpu.VMEM((N,), jnp.float32),
            pltpu.VMEM((C,), jnp.int32),
            pltpu.VMEM((C,), jnp.int32),
            pltpu.VMEM((C + L,), jnp.float32),
            pltpu.VMEM((L,), jnp.int32),
            pltpu.VMEM((L,), jnp.float32),
            pltpu.VMEM((C, D), jnp.float32),
            pltpu.VMEM_SHARED((N, D), jnp.float32),
            pltpu.SemaphoreType.DMA,
        ],
    )
    def sc_edge(hx_hbm, s2_hbm, src_hbm, dst_hbm, zn_hbm, num_out, den_out,
                ea2_t, eb2_t, den_t, sidx, didx, exb, skbuf,
                csbuf, rows, num_sh, sem):
        c = lax.axis_index("c")
        s = lax.axis_index("s")
        wid = s * NC + c

        pltpu.sync_copy(s2_hbm.at[0], ea2_t)
        pltpu.sync_copy(s2_hbm.at[1], eb2_t)

        z16 = jnp.zeros((L,), jnp.float32)

        def zden(i, carry):
            den_t[pl.ds(i * L, L)] = z16
            return carry

        lax.fori_loop(0, N // L, zden, 0)

        @pl.when(s == 0)
        def _():
            pltpu.sync_copy(zn_hbm, num_sh)

        plsc.subcore_barrier()

        lanes = lax.iota(jnp.int32, L)
        nxt = jnp.minimum(lanes + 1, L - 1)
        prv = jnp.maximum(lanes - 1, 0)

        def chunk(i, carry):
            base = wid * EPW + i * C
            pltpu.sync_copy(src_hbm.at[pl.ds(base, C)], sidx)
            pltpu.sync_copy(dst_hbm.at[pl.ds(base, C)], didx)
            gcopy = pltpu.async_copy(hx_hbm.at[sidx], rows, sem)
            for g in range(C // L):
                iv = sidx[pl.ds(g * L, L)]
                dv = didx[pl.ds(g * L, L)]
                p2 = (plsc.load_gather(ea2_t, [iv])
                      * plsc.load_gather(eb2_t, [dv]))
                q = p2 * p2
                ex = jnp.maximum(q * q * p2, p2)
                exb[pl.ds(g * L, L)] = ex
                if g == 0:
                    exb[pl.ds(C, L)] = ex
                sk, sx = plsc.sort_key_val(dv, ex)
                cs = plsc.cumsum(sx)
                skbuf[...] = sk
                csbuf[...] = cs
                sk_next = plsc.load_gather(skbuf, [nxt])
                sk_prev = plsc.load_gather(skbuf, [prv])
                m_end = (sk != sk_next) | (lanes == L - 1)
                r_start = (sk != sk_prev) | (lanes == 0)
                idx_start = plsc.cummax(jnp.where(r_start, lanes, 0))
                cs_prev = plsc.load_gather(
                    csbuf, [jnp.maximum(idx_start - 1, 0)])
                runsum = cs - jnp.where(idx_start > 0, cs_prev, 0.0)
                plsc.addupdate_scatter(den_t, [sk], runsum, mask=m_end)
            gcopy.wait()
            for r in range(C):
                ri = C if r == 0 else r
                scv = plsc.load_gather(
                    exb, [jnp.full((L,), ri, jnp.int32)])
                for j in range(D // L):
                    rows[r, pl.ds(j * L, L)] = rows[r, pl.ds(j * L, L)] * scv
            pltpu.sync_copy(rows, num_sh.at[didx], add=True)
            return carry

        lax.fori_loop(0, NCHUNK, chunk, 0)
        pltpu.sync_copy(den_t, den_out.at[wid])
        plsc.subcore_barrier()

        @pl.when(s == 0)
        def _():
            pltpu.sync_copy(num_sh, num_out.at[c])

    return sc_edge


_sc_edge = _make_sc_edge()


def _elu(v):
    return jnp.where(v > 0.0, v, jnp.exp(jnp.minimum(v, 0.0)) - 1.0)


def _norm_act(num, denp):
    tot = num[0] + num[1]
    den = lax.dot_general(denp, jnp.ones((NW, 1), jnp.float32),
                          (((0,), (0,)), ((), ())),
                          preferred_element_type=jnp.float32,
                          precision=lax.Precision.HIGHEST)
    act = tot / (den + 1e-16)
    return _elu(act)


def _logits(a2, h):
    s2 = lax.dot_general(a2, h, (((1,), (1,)), ((), ())),
                         preferred_element_type=jnp.float32,
                         precision=lax.Precision.HIGHEST)
    return jnp.exp(0.2 * s2)


def _tc_first_body(x_ref, w_ref, a2_ref, hx_ref, s2_ref):
    h = _dot(x_ref[...], w_ref[...])
    hx_ref[...] = h
    s2_ref[...] = _logits(a2_ref[...], h)


def _tc_mid_body(num_ref, denp_ref, w_ref, a2_ref, hx_ref, s2_ref):
    act = _norm_act(num_ref[...], denp_ref[...])
    h = _dot(act, w_ref[...])
    hx_ref[...] = h
    s2_ref[...] = _logits(a2_ref[...], h)


def _tc_cat_body(num_ref, denp_ref, x_ref, wa_ref, wb_ref, a2_ref,
                 hx_ref, s2_ref):
    act = _norm_act(num_ref[...], denp_ref[...])
    h = (_dot(x_ref[...], wa_ref[...])
         + _dot(act, wb_ref[...]))
    hx_ref[...] = h
    s2_ref[...] = _logits(a2_ref[...], h)


_layer_out = (jax.ShapeDtypeStruct((N, D), jnp.float32),
              jax.ShapeDtypeStruct((2, N), jnp.float32))

_tc_first = pl.pallas_call(_tc_first_body, out_shape=_layer_out)
_tc_mid = pl.pallas_call(_tc_mid_body, out_shape=_layer_out)
_tc_cat = pl.pallas_call(_tc_cat_body, out_shape=_layer_out)


def _tc_final_body(num_ref, denp_ref, batch_ref, mf_ref,
                   mw1_ref, mb1_ref, mw2_ref, mb2_ref,
                   wq_ref, wk_ref, wv_ref,
                   pw1_ref, pb1_ref, pw2_ref, pb2_ref, pw3_ref, pb3_ref,
                   pw4_ref, pb4_ref, pw5_ref, pb5_ref,
                   rw1_ref, rb1_ref, rw2_ref, rb2_ref, rw3_ref, rb3_ref,
                   rw4_ref, rb4_ref, rw5_ref, rb5_ref,
                   energy_ref, pi_ref):
    h = _norm_act(num_ref[...], denp_ref[...])
    batch = batch_ref[...]
    onehot = (batch == lax.broadcasted_iota(jnp.int32, (1, B), 1)
              ).astype(jnp.float32)

    def rdot(m, v):
        return lax.dot_general(m, v, (((0,), (0,)), ((), ())),
                               preferred_element_type=jnp.float32,
                               precision=None)

    cnt = rdot(onehot, jnp.ones((N, 1), jnp.float32))
    pooled = rdot(onehot, h) / jnp.maximum(cnt, 1.0)

    z = jax.nn.relu(_dot(pooled, pw1_ref[...]) + pb1_ref[...])
    z = jax.nn.relu(_dot(z, pw2_ref[...]) + pb2_ref[...])
    z = jax.nn.relu(_dot(z, pw3_ref[...]) + pb3_ref[...])
    z = jax.nn.relu(_dot(z, pw4_ref[...]) + pb4_ref[...])
    pi_out = _dot(z, pw5_ref[...]) + pb5_ref[...]

    m1 = jax.nn.relu(_dot(mf_ref[...], mw1_ref[...]) + mb1_ref[...])
    metal = jax.nn.relu(_dot(m1, mw2_ref[...]) + mb2_ref[...])

    q = _dot(metal, wq_ref[...])
    k = _dot(h, wk_ref[...], None)
    v = _dot(h, wv_ref[...], None)
    qn = _dot(onehot, q, None)
    scores = jnp.sum(qn * k, axis=1, keepdims=True) / jnp.sqrt(
        jnp.float32(D))
    mseg = jnp.max(jnp.where(onehot > 0.0, scores, -1e30), axis=0,
                   keepdims=True)
    mn = _dot(onehot, mseg.T, None)
    exn = jnp.exp(scores - mn)
    denb = rdot(onehot, exn)
    dn = _dot(onehot, denb, None)
    w = exn / (dn + 1e-16)
    attn = rdot(onehot * w, v)

    cfeat = jnp.concatenate([pi_out, attn, metal], axis=1)
    r = jax.nn.relu(_dot(cfeat, rw1_ref[...]) + rb1_ref[...])
    r = jax.nn.relu(_dot(r, rw2_ref[...]) + rb2_ref[...])
    r = jax.nn.relu(_dot(r, rw3_ref[...]) + rb3_ref[...])
    r = jax.nn.relu(_dot(r, rw4_ref[...]) + rb4_ref[...])
    energy_ref[...] = _dot(r, rw5_ref[...]) + rb5_ref[...]
    pi_ref[...] = pi_out


_tc_final = pl.pallas_call(
    _tc_final_body,
    out_shape=(jax.ShapeDtypeStruct((B, 1), jnp.float32),
               jax.ShapeDtypeStruct((B, 10), jnp.float32)),
)


def kernel(x, edge_index, batch, metal_features,
           Wn0, Wn, an, Wg0, Wg, ag,
           mw1, mb1, mw2, mb2,
           wq, wk, wv,
           pw1, pb1, pw2, pb2, pw3, pb3, pw4, pb4, pw5, pb5,
           rw1, rb1, rw2, rb2, rw3, rb3, rw4, rb4, rw5, rb5):
    zeros = jnp.zeros((N, D), jnp.float32)

    src_e = edge_index[0]
    dst_e = edge_index[1]
    hx, s2 = _tc_first(x, Wn0, an[0])
    num, denp = _sc_edge(hx, s2, src_e, dst_e, zeros)
    for i in range(3):
        hx, s2 = _tc_mid(num, denp, Wn[i], an[i + 1])
        num, denp = _sc_edge(hx, s2, src_e, dst_e, zeros)
    hx, s2 = _tc_cat(num, denp, x, Wg0[:D], Wg0[D:], ag[0])
    num, denp = _sc_edge(hx, s2, src_e, dst_e, zeros)
    for i in range(3):
        hx, s2 = _tc_mid(num, denp, Wg[i], ag[i + 1])
        num, denp = _sc_edge(hx, s2, src_e, dst_e, zeros)

    energy, pi_out = _tc_final(
        num, denp, batch.reshape(N, 1), metal_features,
        mw1, mb1, mw2, mb2, wq, wk, wv,
        pw1, pb1, pw2, pb2, pw3, pb3, pw4, pb4, pw5, pb5,
        rw1, rb1, rw2, rb2, rw3, rb3, rw4, rb4, rw5, rb5)
    return energy[:, 0], pi_out


# speedup vs baseline: 1.0666x; 1.0000x over previous
import functools

import jax
import jax.numpy as jnp
from jax import lax
from jax.experimental import pallas as pl
from jax.experimental.pallas import tpu as pltpu
from jax.experimental.pallas import tpu_sc as plsc

N = 10000
E = 320000
D = 128
B = 64
NC = 2
NS = 16
NW = NC * NS
EPW = E // NW
C = 80
NCHUNK = EPW // C
L = 16

def _dot(a, b, precision=None):
    return jnp.dot(a, b, preferred_element_type=jnp.float32,
                   precision=precision)


def _make_sc_edge():
    mesh = plsc.VectorSubcoreMesh(core_axis_name="c", subcore_axis_name="s",
                                  num_cores=NC, num_subcores=NS)

    @functools.partial(
        pl.kernel,
        out_type=(jax.ShapeDtypeStruct((NC, N, D), jnp.float32),
                  jax.ShapeDtypeStruct((NW, N), jnp.float32)),
        mesh=mesh,
        compiler_params=pltpu.CompilerParams(needs_layout_passes=False),
        scratch_types=[
            pltpu.VMEM((N,), jnp.float32),
            pltpu.VMEM((N,), jnp.float32),
            pltpu.VMEM((N,), jnp.float32),
            pltpu.VMEM((C,), jnp.int32),
            pltpu.VMEM((C,), jnp.int32),
            pltpu.VMEM((C + L,), jnp.float32),
            pltpu.VMEM((L,), jnp.int32),
            pltpu.VMEM((L,), jnp.float32),
            pltpu.VMEM((C, D), jnp.float32),
            pltpu.VMEM_SHARED((N, D), jnp.float32),
            pltpu.SemaphoreType.DMA,
        ],
    )
    def sc_edge(hx_hbm, s2_hbm, src_hbm, dst_hbm, zn_hbm, num_out, den_out,
                ea2_t, eb2_t, den_t, sidx, didx, exb, skbuf,
                csbuf, rows, num_sh, sem):
        c = lax.axis_index("c")
        s = lax.axis_index("s")
        wid = s * NC + c

        pltpu.sync_copy(s2_hbm.at[0], ea2_t)
        pltpu.sync_copy(s2_hbm.at[1], eb2_t)

        z16 = jnp.zeros((L,), jnp.float32)

        def zden(i, carry):
            den_t[pl.ds(i * L, L)] = z16
            return carry

        lax.fori_loop(0, N // L, zden, 0)

        @pl.when(s < 15)
        def _():
            pltpu.sync_copy(zn_hbm.at[pl.ds(0, 640), :],
                            num_sh.at[pl.ds(s * 640, 640), :])

        @pl.when(s == 15)
        def _():
            pltpu.sync_copy(zn_hbm.at[pl.ds(0, 400), :],
                            num_sh.at[pl.ds(9600, 400), :])

        plsc.subcore_barrier()

        lanes = lax.iota(jnp.int32, L)
        nxt = jnp.minimum(lanes + 1, L - 1)
        prv = jnp.maximum(lanes - 1, 0)

        def chunk(i, carry):
            base = wid * EPW + i * C
            pltpu.sync_copy(src_hbm.at[pl.ds(base, C)], sidx)
            pltpu.sync_copy(dst_hbm.at[pl.ds(base, C)], didx)
            gcopy = pltpu.async_copy(hx_hbm.at[sidx], rows, sem)
            for g in range(C // L):
                iv = sidx[pl.ds(g * L, L)]
                dv = didx[pl.ds(g * L, L)]
                p2 = (plsc.load_gather(ea2_t, [iv])
                      * plsc.load_gather(eb2_t, [dv]))
                q = p2 * p2
                ex = jnp.maximum(q * q * p2, p2)
                exb[pl.ds(g * L, L)] = ex
                if g == 0:
                    exb[pl.ds(C, L)] = ex
                sk, sx = plsc.sort_key_val(dv, ex)
                cs = plsc.cumsum(sx)
                skbuf[...] = sk
                csbuf[...] = cs
                sk_next = plsc.load_gather(skbuf, [nxt])
                sk_prev = plsc.load_gather(skbuf, [prv])
                m_end = (sk != sk_next) | (lanes == L - 1)
                r_start = (sk != sk_prev) | (lanes == 0)
                idx_start = plsc.cummax(jnp.where(r_start, lanes, 0))
                cs_prev = plsc.load_gather(
                    csbuf, [jnp.maximum(idx_start - 1, 0)])
                runsum = cs - jnp.where(idx_start > 0, cs_prev, 0.0)
                plsc.addupdate_scatter(den_t, [sk], runsum, mask=m_end)
            gcopy.wait()
            for r in range(C):
                ri = C if r == 0 else r
                scv = plsc.load_gather(
                    exb, [jnp.full((L,), ri, jnp.int32)])
                for j in range(D // L):
                    rows[r, pl.ds(j * L, L)] = rows[r, pl.ds(j * L, L)] * scv
            pltpu.sync_copy(rows, num_sh.at[didx], add=True)
            return carry

        lax.fori_loop(0, NCHUNK, chunk, 0)
        pltpu.sync_copy(den_t, den_out.at[wid])
        plsc.subcore_barrier()

        @pl.when(s < 15)
        def _():
            pltpu.sync_copy(num_sh.at[pl.ds(s * 640, 640), :],
                            num_out.at[c, pl.ds(s * 640, 640), :])

        @pl.when(s == 15)
        def _():
            pltpu.sync_copy(num_sh.at[pl.ds(9600, 400), :],
                            num_out.at[c, pl.ds(9600, 400), :])

    return sc_edge


_sc_edge = _make_sc_edge()


def _elu(v):
    return jnp.where(v > 0.0, v, jnp.exp(jnp.minimum(v, 0.0)) - 1.0)


def _norm_act(num, denp):
    tot = num[0] + num[1]
    den = lax.dot_general(denp, jnp.ones((NW, 1), jnp.float32),
                          (((0,), (0,)), ((), ())),
                          preferred_element_type=jnp.float32,
                          precision=lax.Precision.HIGHEST)
    act = tot / (den + 1e-16)
    return _elu(act)


def _logits(a2, h):
    s2 = lax.dot_general(a2, h, (((1,), (1,)), ((), ())),
                         preferred_element_type=jnp.float32,
                         precision=lax.Precision.HIGHEST)
    return jnp.exp(0.2 * s2)


def _tc_first_body(x_ref, w_ref, a2_ref, hx_ref, s2_ref):
    h = _dot(x_ref[...], w_ref[...])
    hx_ref[...] = h
    s2_ref[...] = _logits(a2_ref[...], h)


def _tc_mid_body(num_ref, denp_ref, w_ref, a2_ref, hx_ref, s2_ref):
    act = _norm_act(num_ref[...], denp_ref[...])
    h = _dot(act, w_ref[...])
    hx_ref[...] = h
    s2_ref[...] = _logits(a2_ref[...], h)


def _tc_cat_body(num_ref, denp_ref, x_ref, wa_ref, wb_ref, a2_ref,
                 hx_ref, s2_ref):
    act = _norm_act(num_ref[...], denp_ref[...])
    h = (_dot(x_ref[...], wa_ref[...])
         + _dot(act, wb_ref[...]))
    hx_ref[...] = h
    s2_ref[...] = _logits(a2_ref[...], h)


_layer_out = (jax.ShapeDtypeStruct((N, D), jnp.float32),
              jax.ShapeDtypeStruct((2, N), jnp.float32))

_tc_first = pl.pallas_call(_tc_first_body, out_shape=_layer_out)
_tc_mid = pl.pallas_call(_tc_mid_body, out_shape=_layer_out)
_tc_cat = pl.pallas_call(_tc_cat_body, out_shape=_layer_out)


def _tc_final_body(num_ref, denp_ref, batch_ref, mf_ref,
                   mw1_ref, mb1_ref, mw2_ref, mb2_ref,
                   wq_ref, wk_ref, wv_ref,
                   pw1_ref, pb1_ref, pw2_ref, pb2_ref, pw3_ref, pb3_ref,
                   pw4_ref, pb4_ref, pw5_ref, pb5_ref,
                   rw1_ref, rb1_ref, rw2_ref, rb2_ref, rw3_ref, rb3_ref,
                   rw4_ref, rb4_ref, rw5_ref, rb5_ref,
                   energy_ref, pi_ref):
    h = _norm_act(num_ref[...], denp_ref[...])
    batch = batch_ref[...]
    onehot = (batch == lax.broadcasted_iota(jnp.int32, (1, B), 1)
              ).astype(jnp.float32)

    def rdot(m, v):
        return lax.dot_general(m, v, (((0,), (0,)), ((), ())),
                               preferred_element_type=jnp.float32,
                               precision=None)

    cnt = rdot(onehot, jnp.ones((N, 1), jnp.float32))
    pooled = rdot(onehot, h) / jnp.maximum(cnt, 1.0)

    z = jax.nn.relu(_dot(pooled, pw1_ref[...]) + pb1_ref[...])
    z = jax.nn.relu(_dot(z, pw2_ref[...]) + pb2_ref[...])
    z = jax.nn.relu(_dot(z, pw3_ref[...]) + pb3_ref[...])
    z = jax.nn.relu(_dot(z, pw4_ref[...]) + pb4_ref[...])
    pi_out = _dot(z, pw5_ref[...]) + pb5_ref[...]

    m1 = jax.nn.relu(_dot(mf_ref[...], mw1_ref[...]) + mb1_ref[...])
    metal = jax.nn.relu(_dot(m1, mw2_ref[...]) + mb2_ref[...])

    q = _dot(metal, wq_ref[...])
    k = _dot(h, wk_ref[...], None)
    v = _dot(h, wv_ref[...], None)
    qn = _dot(onehot, q, None)
    scores = jnp.sum(qn * k, axis=1, keepdims=True) / jnp.sqrt(
        jnp.float32(D))
    mseg = jnp.max(jnp.where(onehot > 0.0, scores, -1e30), axis=0,
                   keepdims=True)
    mn = _dot(onehot, mseg.T, None)
    exn = jnp.exp(scores - mn)
    denb = rdot(onehot, exn)
    dn = _dot(onehot, denb, None)
    w = exn / (dn + 1e-16)
    attn = rdot(onehot * w, v)

    cfeat = jnp.concatenate([pi_out, attn, metal], axis=1)
    r = jax.nn.relu(_dot(cfeat, rw1_ref[...]) + rb1_ref[...])
    r = jax.nn.relu(_dot(r, rw2_ref[...]) + rb2_ref[...])
    r = jax.nn.relu(_dot(r, rw3_ref[...]) + rb3_ref[...])
    r = jax.nn.relu(_dot(r, rw4_ref[...]) + rb4_ref[...])
    energy_ref[...] = _dot(r, rw5_ref[...]) + rb5_ref[...]
    pi_ref[...] = pi_out


_tc_final = pl.pallas_call(
    _tc_final_body,
    out_shape=(jax.ShapeDtypeStruct((B, 1), jnp.float32),
               jax.ShapeDtypeStruct((B, 10), jnp.float32)),
)


def kernel(x, edge_index, batch, metal_features,
           Wn0, Wn, an, Wg0, Wg, ag,
           mw1, mb1, mw2, mb2,
           wq, wk, wv,
           pw1, pb1, pw2, pb2, pw3, pb3, pw4, pb4, pw5, pb5,
           rw1, rb1, rw2, rb2, rw3, rb3, rw4, rb4, rw5, rb5):
    zeros = jnp.zeros((N, D), jnp.float32)

    src_e = edge_index[0]
    dst_e = edge_index[1]
    hx, s2 = _tc_first(x, Wn0, an[0])
    num, denp = _sc_edge(hx, s2, src_e, dst_e, zeros)
    for i in range(3):
        hx, s2 = _tc_mid(num, denp, Wn[i], an[i + 1])
        num, denp = _sc_edge(hx, s2, src_e, dst_e, zeros)
    hx, s2 = _tc_cat(num, denp, x, Wg0[:D], Wg0[D:], ag[0])
    num, denp = _sc_edge(hx, s2, src_e, dst_e, zeros)
    for i in range(3):
        hx, s2 = _tc_mid(num, denp, Wg[i], ag[i + 1])
        num, denp = _sc_edge(hx, s2, src_e, dst_e, zeros)

    energy, pi_out = _tc_final(
        num, denp, batch.reshape(N, 1), metal_features,
        mw1, mb1, mw2, mb2, wq, wk, wv,
        pw1, pb1, pw2, pb2, pw3, pb3, pw4, pb4, pw5, pb5,
        rw1, rb1, rw2, rb2, rw3, rb3, rw4, rb4, rw5, rb5)
    return energy[:, 0], pi_out
